# baseline (speedup 1.0000x reference)
import functools

import jax
import jax.numpy as jnp
from jax import lax
from jax.experimental import pallas as pl
from jax.experimental.pallas import tpu as pltpu
from jax.experimental.pallas import tpu_sc as plsc

N = 10000
E = 320000
D = 128
NC = 2
NS = 16
NW = NC * NS
EPW = E // NW
CH = 128
NCHUNK = -(-EPW // CH)
EPWP = NCHUNK * CH
N2 = 10240
TRASH = N2 - 1
RPT = N2 // NS
ZR = 16
DEGW = 16
BLK = 400
GRID = N // BLK

_mesh = plsc.VectorSubcoreMesh(
    core_axis_name="c", subcore_axis_name="s", num_cores=NC, num_subcores=NS
)


def _unpack_chunk(pk_v, jc, st_s, st_d):
    def u(k, carry):
        v = pk_v[jc, 0, pl.ds(k * 16, 16)]
        if st_s is not None:
            st_s[pl.ds(k * 16, 16)] = jnp.bitwise_and(v, 0xFFFF)
        st_d[pl.ds(k * 16, 16)] = lax.shift_right_logical(v, 16)
        return carry

    lax.fori_loop(0, CH // 16, u, 0)


@functools.partial(
    pl.kernel,
    out_type=jax.ShapeDtypeStruct((NC, N2, DEGW), jnp.float32),
    mesh=_mesh,
    scratch_types=[
        pltpu.VMEM((NCHUNK, 1, CH), jnp.int32),
        pltpu.VMEM((CH,), jnp.int32),
        pltpu.VMEM((CH,), jnp.int32),
        pltpu.VMEM((CH, DEGW), jnp.float32),
        pltpu.VMEM((ZR, DEGW), jnp.float32),
        pltpu.VMEM_SHARED((N2, DEGW), jnp.float32),
        pltpu.SemaphoreType.DMA,
        pltpu.SemaphoreType.DMA,
    ],
)
def _sc_deg(pk_hbm, out_hbm, pk_v, std_a, std_b, ones_v, zer_v, acc, sem_a, sem_b):
    c = lax.axis_index("c")
    s = lax.axis_index("s")
    w = c * NS + s

    pltpu.async_copy(pk_hbm.at[w], pk_v, sem_a)

    def fill_ones(i, carry):
        ones_v[i, :] = jnp.ones((16,), jnp.float32)
        return carry

    lax.fori_loop(0, CH, fill_ones, 0)

    def fill_zero(i, carry):
        zer_v[i, :] = jnp.zeros((16,), jnp.float32)
        return carry

    lax.fori_loop(0, ZR, fill_zero, 0)

    for k in range(RPT // ZR):
        pltpu.sync_copy(zer_v, acc.at[pl.ds(s * RPT + k * ZR, ZR)])

    pltpu.make_async_copy(pk_hbm.at[w], pk_v, sem_a).wait()
    plsc.subcore_barrier()

    _unpack_chunk(pk_v, 0, None, std_a)
    pltpu.async_copy(ones_v, acc.at[std_a], sem_a, add=True)
    _unpack_chunk(pk_v, 1, None, std_b)
    pltpu.async_copy(ones_v, acc.at[std_b], sem_b, add=True)

    def body(jj, carry):
        j = 2 * jj
        pltpu.make_async_copy(ones_v, acc.at[std_a], sem_a).wait()
        _unpack_chunk(pk_v, j + 2, None, std_a)
        pltpu.async_copy(ones_v, acc.at[std_a], sem_a, add=True)
        pltpu.make_async_copy(ones_v, acc.at[std_b], sem_b).wait()
        _unpack_chunk(pk_v, j + 3, None, std_b)
        pltpu.async_copy(ones_v, acc.at[std_b], sem_b, add=True)
        return carry

    lax.fori_loop(0, (NCHUNK - 3) // 2, body, 0)
    pltpu.make_async_copy(ones_v, acc.at[std_a], sem_a).wait()
    _unpack_chunk(pk_v, NCHUNK - 1, None, std_a)
    pltpu.async_copy(ones_v, acc.at[std_a], sem_a, add=True)
    pltpu.make_async_copy(ones_v, acc.at[std_a], sem_a).wait()
    pltpu.make_async_copy(ones_v, acc.at[std_b], sem_b).wait()
    plsc.subcore_barrier()
    pltpu.sync_copy(acc.at[pl.ds(s * RPT, RPT)], out_hbm.at[c, pl.ds(s * RPT, RPT)])


@functools.partial(
    pl.kernel,
    out_type=jax.ShapeDtypeStruct((NC, N2, D), jnp.float32),
    mesh=_mesh,
    scratch_types=[
        pltpu.VMEM((NCHUNK, 1, CH), jnp.int32),
        pltpu.VMEM((NCHUNK, 1, CH), jnp.int32),
        pltpu.VMEM((CH, D), jnp.float32),
        pltpu.VMEM((ZR, D), jnp.float32),
        pltpu.VMEM_SHARED((N2, D), jnp.float32),
        pltpu.SemaphoreType.DMA,
        pltpu.SemaphoreType.DMA,
    ],
)
def _sc_agg(g_hbm, src4_hbm, dst4_hbm, out_hbm, idxs_v, idxd_v,
            rows_v, zer_v, acc, sem_g, sem_i):
    c = lax.axis_index("c")
    s = lax.axis_index("s")
    w = c * NS + s

    pltpu.async_copy(src4_hbm.at[pl.ds(w * NCHUNK, NCHUNK)], idxs_v, sem_i)
    pltpu.async_copy(dst4_hbm.at[pl.ds(w * NCHUNK, NCHUNK)], idxd_v, sem_i)

    def fill_zero(i, carry):
        r = i // (D // 16)
        k = i - r * (D // 16)
        zer_v[r, pl.ds(k * 16, 16)] = jnp.zeros((16,), jnp.float32)
        return carry

    lax.fori_loop(0, ZR * (D // 16), fill_zero, 0)

    for k in range(RPT // ZR):
        pltpu.sync_copy(zer_v, acc.at[pl.ds(s * RPT + k * ZR, ZR)])

    pltpu.make_async_copy(src4_hbm.at[pl.ds(w * NCHUNK, NCHUNK)], idxs_v, sem_i).wait()
    pltpu.make_async_copy(dst4_hbm.at[pl.ds(w * NCHUNK, NCHUNK)], idxd_v, sem_i).wait()
    plsc.subcore_barrier()

    def body(j, carry):
        pltpu.async_copy(g_hbm.at[idxs_v.at[j, 0]], rows_v, sem_g).wait()
        pltpu.sync_copy(rows_v, acc.at[idxd_v.at[j, 0]], add=True)
        return carry

    lax.fori_loop(0, NCHUNK, body, 0)
    plsc.subcore_barrier()
    pltpu.sync_copy(acc.at[pl.ds(s * RPT, RPT)], out_hbm.at[c, pl.ds(s * RPT, RPT)])


def _dis(deg_ref):
    return lax.rsqrt(1.0 + deg_ref[0, :, 0:1] + deg_ref[1, :, 0:1])


def _tc_g1_body(x_ref, w_ref, deg_ref, o_ref):
    h = jnp.dot(x_ref[...], w_ref[...], preferred_element_type=jnp.float32)
    o_ref[...] = h * _dis(deg_ref)


def _tc_g2_body(p_ref, g_ref, deg_ref, b_ref, w_ref, o_ref):
    dis = _dis(deg_ref)
    ssum = p_ref[0] + p_ref[1] + g_ref[...]
    h = jnp.maximum(dis * ssum + b_ref[...], 0.0)
    o_ref[...] = jnp.dot(h, w_ref[...], preferred_element_type=jnp.float32) * dis


def _tc_out_body(p_ref, g_ref, deg_ref, b_ref, o_ref):
    dis = _dis(deg_ref)
    o_ref[...] = dis * (p_ref[0] + p_ref[1] + g_ref[...]) + b_ref[...]


_row_spec = pl.BlockSpec((BLK, D), lambda i: (i, 0))
_w_spec = pl.BlockSpec((D, D), lambda i: (0, 0))
_deg_spec = pl.BlockSpec((NC, BLK, DEGW), lambda i: (0, i, 0))
_p_spec = pl.BlockSpec((NC, BLK, D), lambda i: (0, i, 0))
_b_spec = pl.BlockSpec((1, D), lambda i: (0, 0))

_g1_call = pl.pallas_call(
    _tc_g1_body,
    grid=(GRID,),
    in_specs=[_row_spec, _w_spec, _deg_spec],
    out_specs=_row_spec,
    out_shape=jax.ShapeDtypeStruct((N, D), jnp.float32),
)

_g2_call = pl.pallas_call(
    _tc_g2_body,
    grid=(GRID,),
    in_specs=[_p_spec, _row_spec, _deg_spec, _b_spec, _w_spec],
    out_specs=_row_spec,
    out_shape=jax.ShapeDtypeStruct((N, D), jnp.float32),
)

_out_call = pl.pallas_call(
    _tc_out_body,
    grid=(GRID,),
    in_specs=[_p_spec, _row_spec, _deg_spec, _b_spec],
    out_specs=_row_spec,
    out_shape=jax.ShapeDtypeStruct((N, D), jnp.float32),
)


def kernel(x, edge_index, W1, b1, W2, b2):
    packed = jnp.bitwise_or(
        edge_index[0], jnp.left_shift(edge_index[1], 16)
    ).reshape(NW, EPW)
    padv = jnp.full((NW, EPWP - EPW), TRASH << 16, dtype=jnp.int32)
    pk4 = jnp.concatenate([packed, padv], axis=1).reshape(NW, NCHUNK, 1, CH)
    pads = jnp.zeros((NW, EPWP - EPW), dtype=jnp.int32)
    padd = jnp.full((NW, EPWP - EPW), TRASH, dtype=jnp.int32)
    src4 = jnp.concatenate(
        [edge_index[0].reshape(NW, EPW), pads], axis=1
    ).reshape(NW * NCHUNK, 1, CH)
    dst4 = jnp.concatenate(
        [edge_index[1].reshape(NW, EPW), padd], axis=1
    ).reshape(NW * NCHUNK, 1, CH)
    b1r = b1.reshape(1, D)
    b2r = b2.reshape(1, D)

    deg16 = _sc_deg(pk4)
    g1 = _g1_call(x, W1, deg16)
    p1 = _sc_agg(g1, src4, dst4)
    g2 = _g2_call(p1, g1, deg16, b1r, W2)
    p2 = _sc_agg(g2, src4, dst4)
    out = _out_call(p2, g2, deg16, b2r)
    return out


# speedup vs baseline: 1.3638x; 1.3638x over previous
import functools

import jax
import jax.numpy as jnp
from jax import lax
from jax.experimental import pallas as pl
from jax.experimental.pallas import tpu as pltpu
from jax.experimental.pallas import tpu_sc as plsc

N = 10000
E = 320000
D = 128
NC = 2
NS = 16
NW = NC * NS
EPW = E // NW
CH = 80
NCHUNK = EPW // CH
CHD = 128
NCHUNKD = -(-EPW // CHD)
EPWPD = NCHUNKD * CHD
N2 = 10240
TRASH = N2 - 1
RPT = N2 // NS
ZR = 16
DEGW = 16
BLK = 400
GRID = N // BLK

_mesh = plsc.VectorSubcoreMesh(
    core_axis_name="c", subcore_axis_name="s", num_cores=NC, num_subcores=NS
)


def _unpack_chunk(pk_v, jc, st_s, st_d):
    def u(k, carry):
        v = pk_v[jc, 0, pl.ds(k * 16, 16)]
        if st_s is not None:
            st_s[pl.ds(k * 16, 16)] = jnp.bitwise_and(v, 0xFFFF)
        st_d[pl.ds(k * 16, 16)] = lax.shift_right_logical(v, 16)
        return carry

    lax.fori_loop(0, CHD // 16, u, 0)


@functools.partial(
    pl.kernel,
    out_type=jax.ShapeDtypeStruct((NC, N2, DEGW), jnp.float32),
    mesh=_mesh,
    scratch_types=[
        pltpu.VMEM((NCHUNKD, 1, CHD), jnp.int32),
        pltpu.VMEM((CHD,), jnp.int32),
        pltpu.VMEM((CHD,), jnp.int32),
        pltpu.VMEM((CHD, DEGW), jnp.float32),
        pltpu.VMEM((ZR, DEGW), jnp.float32),
        pltpu.VMEM_SHARED((N2, DEGW), jnp.float32),
        pltpu.SemaphoreType.DMA,
        pltpu.SemaphoreType.DMA,
    ],
)
def _sc_deg(pk_hbm, out_hbm, pk_v, std_a, std_b, ones_v, zer_v, acc, sem_a, sem_b):
    c = lax.axis_index("c")
    s = lax.axis_index("s")
    w = c * NS + s

    pltpu.async_copy(pk_hbm.at[w], pk_v, sem_a)

    def fill_ones(i, carry):
        ones_v[i, :] = jnp.ones((16,), jnp.float32)
        return carry

    lax.fori_loop(0, CHD, fill_ones, 0)

    def fill_zero(i, carry):
        zer_v[i, :] = jnp.zeros((16,), jnp.float32)
        return carry

    lax.fori_loop(0, ZR, fill_zero, 0)

    for k in range(RPT // ZR):
        pltpu.sync_copy(zer_v, acc.at[pl.ds(s * RPT + k * ZR, ZR)])

    pltpu.make_async_copy(pk_hbm.at[w], pk_v, sem_a).wait()
    plsc.subcore_barrier()

    _unpack_chunk(pk_v, 0, None, std_a)
    pltpu.async_copy(ones_v, acc.at[std_a], sem_a, add=True)
    _unpack_chunk(pk_v, 1, None, std_b)
    pltpu.async_copy(ones_v, acc.at[std_b], sem_b, add=True)

    def body(jj, carry):
        j = 2 * jj
        pltpu.make_async_copy(ones_v, acc.at[std_a], sem_a).wait()
        _unpack_chunk(pk_v, j + 2, None, std_a)
        pltpu.async_copy(ones_v, acc.at[std_a], sem_a, add=True)
        pltpu.make_async_copy(ones_v, acc.at[std_b], sem_b).wait()
        _unpack_chunk(pk_v, j + 3, None, std_b)
        pltpu.async_copy(ones_v, acc.at[std_b], sem_b, add=True)
        return carry

    lax.fori_loop(0, (NCHUNKD - 3) // 2, body, 0)
    pltpu.make_async_copy(ones_v, acc.at[std_a], sem_a).wait()
    _unpack_chunk(pk_v, NCHUNKD - 1, None, std_a)
    pltpu.async_copy(ones_v, acc.at[std_a], sem_a, add=True)
    pltpu.make_async_copy(ones_v, acc.at[std_a], sem_a).wait()
    pltpu.make_async_copy(ones_v, acc.at[std_b], sem_b).wait()
    plsc.subcore_barrier()
    pltpu.sync_copy(acc.at[pl.ds(s * RPT, RPT)], out_hbm.at[c, pl.ds(s * RPT, RPT)])


@functools.partial(
    pl.kernel,
    out_type=jax.ShapeDtypeStruct((NC, N2, D), jnp.float32),
    mesh=_mesh,
    scratch_types=[
        pltpu.VMEM((NCHUNK, 1, CH), jnp.int32),
        pltpu.VMEM((NCHUNK, 1, CH), jnp.int32),
        pltpu.VMEM((CH, D), jnp.float32),
        pltpu.VMEM((ZR, D), jnp.float32),
        pltpu.VMEM_SHARED((N2, D), jnp.float32),
        pltpu.SemaphoreType.DMA,
        pltpu.SemaphoreType.DMA,
    ],
)
def _sc_agg(g_hbm, src4_hbm, dst4_hbm, out_hbm, idxs_v, idxd_v,
            rows_v, zer_v, acc, sem_g, sem_i):
    c = lax.axis_index("c")
    s = lax.axis_index("s")
    w = c * NS + s

    pltpu.async_copy(src4_hbm.at[pl.ds(w * NCHUNK, NCHUNK)], idxs_v, sem_i)
    pltpu.async_copy(dst4_hbm.at[pl.ds(w * NCHUNK, NCHUNK)], idxd_v, sem_i)

    def fill_zero(i, carry):
        r = i // (D // 16)
        k = i - r * (D // 16)
        zer_v[r, pl.ds(k * 16, 16)] = jnp.zeros((16,), jnp.float32)
        return carry

    lax.fori_loop(0, ZR * (D // 16), fill_zero, 0)

    for k in range(RPT // ZR):
        pltpu.sync_copy(zer_v, acc.at[pl.ds(s * RPT + k * ZR, ZR)])

    pltpu.make_async_copy(src4_hbm.at[pl.ds(w * NCHUNK, NCHUNK)], idxs_v, sem_i).wait()
    pltpu.make_async_copy(dst4_hbm.at[pl.ds(w * NCHUNK, NCHUNK)], idxd_v, sem_i).wait()
    plsc.subcore_barrier()

    def body(j, carry):
        pltpu.async_copy(g_hbm.at[idxs_v.at[j, 0]], rows_v, sem_g).wait()
        pltpu.sync_copy(rows_v, acc.at[idxd_v.at[j, 0]], add=True)
        return carry

    lax.fori_loop(0, NCHUNK, body, 0)
    plsc.subcore_barrier()
    pltpu.sync_copy(acc.at[pl.ds(s * RPT, RPT)], out_hbm.at[c, pl.ds(s * RPT, RPT)])


def _dis(deg_ref):
    return lax.rsqrt(1.0 + deg_ref[0, :, 0:1] + deg_ref[1, :, 0:1])


def _tc_g1_body(x_ref, w_ref, deg_ref, o_ref):
    h = jnp.dot(x_ref[...], w_ref[...], preferred_element_type=jnp.float32)
    o_ref[...] = h * _dis(deg_ref)


def _tc_g2_body(p_ref, g_ref, deg_ref, b_ref, w_ref, o_ref):
    dis = _dis(deg_ref)
    ssum = p_ref[0] + p_ref[1] + g_ref[...]
    h = jnp.maximum(dis * ssum + b_ref[...], 0.0)
    o_ref[...] = jnp.dot(h, w_ref[...], preferred_element_type=jnp.float32) * dis


def _tc_out_body(p_ref, g_ref, deg_ref, b_ref, o_ref):
    dis = _dis(deg_ref)
    o_ref[...] = dis * (p_ref[0] + p_ref[1] + g_ref[...]) + b_ref[...]


_row_spec = pl.BlockSpec((BLK, D), lambda i: (i, 0))
_w_spec = pl.BlockSpec((D, D), lambda i: (0, 0))
_deg_spec = pl.BlockSpec((NC, BLK, DEGW), lambda i: (0, i, 0))
_p_spec = pl.BlockSpec((NC, BLK, D), lambda i: (0, i, 0))
_b_spec = pl.BlockSpec((1, D), lambda i: (0, 0))

_g1_call = pl.pallas_call(
    _tc_g1_body,
    grid=(GRID,),
    in_specs=[_row_spec, _w_spec, _deg_spec],
    out_specs=_row_spec,
    out_shape=jax.ShapeDtypeStruct((N, D), jnp.float32),
)

_g2_call = pl.pallas_call(
    _tc_g2_body,
    grid=(GRID,),
    in_specs=[_p_spec, _row_spec, _deg_spec, _b_spec, _w_spec],
    out_specs=_row_spec,
    out_shape=jax.ShapeDtypeStruct((N, D), jnp.float32),
)

_out_call = pl.pallas_call(
    _tc_out_body,
    grid=(GRID,),
    in_specs=[_p_spec, _row_spec, _deg_spec, _b_spec],
    out_specs=_row_spec,
    out_shape=jax.ShapeDtypeStruct((N, D), jnp.float32),
)


def kernel(x, edge_index, W1, b1, W2, b2):
    packed = jnp.bitwise_or(
        edge_index[0], jnp.left_shift(edge_index[1], 16)
    ).reshape(NW, EPW)
    padv = jnp.full((NW, EPWPD - EPW), TRASH << 16, dtype=jnp.int32)
    pk4 = jnp.concatenate([packed, padv], axis=1).reshape(NW, NCHUNKD, 1, CHD)
    src4 = edge_index[0].reshape(NW * NCHUNK, 1, CH)
    dst4 = edge_index[1].reshape(NW * NCHUNK, 1, CH)
    b1r = b1.reshape(1, D)
    b2r = b2.reshape(1, D)

    deg16 = _sc_deg(pk4)
    g1 = _g1_call(x, W1, deg16)
    p1 = _sc_agg(g1, src4, dst4)
    g2 = _g2_call(p1, g1, deg16, b1r, W2)
    p2 = _sc_agg(g2, src4, dst4)
    out = _out_call(p2, g2, deg16, b2r)
    return out


# speedup vs baseline: 1.4590x; 1.0698x over previous
import functools

import jax
import jax.numpy as jnp
from jax import lax
from jax.experimental import pallas as pl
from jax.experimental.pallas import tpu as pltpu
from jax.experimental.pallas import tpu_sc as plsc

N = 10000
E = 320000
D = 128
NC = 2
NS = 16
NW = NC * NS
EPW = E // NW
CH = 100
NCHUNK = EPW // CH
CHD = 128
NCHUNKD = -(-EPW // CHD)
EPWPD = NCHUNKD * CHD
N2 = 10240
TRASH = N2 - 1
RPT = N2 // NS
ZR = 16
DEGW = 16
BLK = 400
GRID = N // BLK

_mesh = plsc.VectorSubcoreMesh(
    core_axis_name="c", subcore_axis_name="s", num_cores=NC, num_subcores=NS
)


def _unpack_chunk(pk_v, jc, st_s, st_d):
    def u(k, carry):
        v = pk_v[jc, 0, pl.ds(k * 16, 16)]
        if st_s is not None:
            st_s[pl.ds(k * 16, 16)] = jnp.bitwise_and(v, 0xFFFF)
        st_d[pl.ds(k * 16, 16)] = lax.shift_right_logical(v, 16)
        return carry

    lax.fori_loop(0, CHD // 16, u, 0)


@functools.partial(
    pl.kernel,
    out_type=jax.ShapeDtypeStruct((NC, N2, DEGW), jnp.float32),
    mesh=_mesh,
    scratch_types=[
        pltpu.VMEM((NCHUNKD, 1, CHD), jnp.int32),
        pltpu.VMEM((CHD,), jnp.int32),
        pltpu.VMEM((CHD,), jnp.int32),
        pltpu.VMEM((CHD, DEGW), jnp.float32),
        pltpu.VMEM((ZR, DEGW), jnp.float32),
        pltpu.VMEM_SHARED((N2, DEGW), jnp.float32),
        pltpu.SemaphoreType.DMA,
        pltpu.SemaphoreType.DMA,
    ],
)
def _sc_deg(pk_hbm, out_hbm, pk_v, std_a, std_b, ones_v, zer_v, acc, sem_a, sem_b):
    c = lax.axis_index("c")
    s = lax.axis_index("s")
    w = c * NS + s

    pltpu.async_copy(pk_hbm.at[w], pk_v, sem_a)

    def fill_ones(i, carry):
        ones_v[i, :] = jnp.ones((16,), jnp.float32)
        return carry

    lax.fori_loop(0, CHD, fill_ones, 0)

    def fill_zero(i, carry):
        zer_v[i, :] = jnp.zeros((16,), jnp.float32)
        return carry

    lax.fori_loop(0, ZR, fill_zero, 0)

    for k in range(RPT // ZR):
        pltpu.sync_copy(zer_v, acc.at[pl.ds(s * RPT + k * ZR, ZR)])

    pltpu.make_async_copy(pk_hbm.at[w], pk_v, sem_a).wait()
    plsc.subcore_barrier()

    _unpack_chunk(pk_v, 0, None, std_a)
    pltpu.async_copy(ones_v, acc.at[std_a], sem_a, add=True)
    _unpack_chunk(pk_v, 1, None, std_b)
    pltpu.async_copy(ones_v, acc.at[std_b], sem_b, add=True)

    def body(jj, carry):
        j = 2 * jj
        pltpu.make_async_copy(ones_v, acc.at[std_a], sem_a).wait()
        _unpack_chunk(pk_v, j + 2, None, std_a)
        pltpu.async_copy(ones_v, acc.at[std_a], sem_a, add=True)
        pltpu.make_async_copy(ones_v, acc.at[std_b], sem_b).wait()
        _unpack_chunk(pk_v, j + 3, None, std_b)
        pltpu.async_copy(ones_v, acc.at[std_b], sem_b, add=True)
        return carry

    lax.fori_loop(0, (NCHUNKD - 3) // 2, body, 0)
    pltpu.make_async_copy(ones_v, acc.at[std_a], sem_a).wait()
    _unpack_chunk(pk_v, NCHUNKD - 1, None, std_a)
    pltpu.async_copy(ones_v, acc.at[std_a], sem_a, add=True)
    pltpu.make_async_copy(ones_v, acc.at[std_a], sem_a).wait()
    pltpu.make_async_copy(ones_v, acc.at[std_b], sem_b).wait()
    plsc.subcore_barrier()
    pltpu.sync_copy(acc.at[pl.ds(s * RPT, RPT)], out_hbm.at[c, pl.ds(s * RPT, RPT)])


@functools.partial(
    pl.kernel,
    out_type=jax.ShapeDtypeStruct((NC, N2, D), jnp.float32),
    mesh=_mesh,
    scratch_types=[
        pltpu.VMEM((NCHUNK, 1, CH), jnp.int32),
        pltpu.VMEM((NCHUNK, 1, CH), jnp.int32),
        pltpu.VMEM((CH, D), jnp.float32),
        pltpu.VMEM((ZR, D), jnp.float32),
        pltpu.VMEM_SHARED((N2, D), jnp.float32),
        pltpu.SemaphoreType.DMA,
        pltpu.SemaphoreType.DMA,
    ],
)
def _sc_agg(g_hbm, src4_hbm, dst4_hbm, out_hbm, idxs_v, idxd_v,
            rows_v, zer_v, acc, sem_g, sem_i):
    c = lax.axis_index("c")
    s = lax.axis_index("s")
    w = c * NS + s

    pltpu.async_copy(src4_hbm.at[pl.ds(w * NCHUNK, NCHUNK)], idxs_v, sem_i)
    pltpu.async_copy(dst4_hbm.at[pl.ds(w * NCHUNK, NCHUNK)], idxd_v, sem_i)

    def fill_zero(i, carry):
        r = i // (D // 16)
        k = i - r * (D // 16)
        zer_v[r, pl.ds(k * 16, 16)] = jnp.zeros((16,), jnp.float32)
        return carry

    lax.fori_loop(0, ZR * (D // 16), fill_zero, 0)

    for k in range(RPT // ZR):
        pltpu.sync_copy(zer_v, acc.at[pl.ds(s * RPT + k * ZR, ZR)])

    pltpu.make_async_copy(src4_hbm.at[pl.ds(w * NCHUNK, NCHUNK)], idxs_v, sem_i).wait()
    pltpu.make_async_copy(dst4_hbm.at[pl.ds(w * NCHUNK, NCHUNK)], idxd_v, sem_i).wait()
    plsc.subcore_barrier()

    def body(j, carry):
        pltpu.async_copy(g_hbm.at[idxs_v.at[j, 0]], rows_v, sem_g).wait()
        pltpu.sync_copy(rows_v, acc.at[idxd_v.at[j, 0]], add=True)
        return carry

    lax.fori_loop(0, NCHUNK, body, 0)
    plsc.subcore_barrier()
    pltpu.sync_copy(acc.at[pl.ds(s * RPT, RPT)], out_hbm.at[c, pl.ds(s * RPT, RPT)])


def _dis(deg_ref):
    return lax.rsqrt(1.0 + deg_ref[0, :, 0:1] + deg_ref[1, :, 0:1])


def _tc_g1_body(x_ref, w_ref, deg_ref, o_ref):
    h = jnp.dot(x_ref[...], w_ref[...], preferred_element_type=jnp.float32)
    o_ref[...] = h * _dis(deg_ref)


def _tc_g2_body(p_ref, g_ref, deg_ref, b_ref, w_ref, o_ref):
    dis = _dis(deg_ref)
    ssum = p_ref[0] + p_ref[1] + g_ref[...]
    h = jnp.maximum(dis * ssum + b_ref[...], 0.0)
    o_ref[...] = jnp.dot(h, w_ref[...], preferred_element_type=jnp.float32) * dis


def _tc_out_body(p_ref, g_ref, deg_ref, b_ref, o_ref):
    dis = _dis(deg_ref)
    o_ref[...] = dis * (p_ref[0] + p_ref[1] + g_ref[...]) + b_ref[...]


_row_spec = pl.BlockSpec((BLK, D), lambda i: (i, 0))
_w_spec = pl.BlockSpec((D, D), lambda i: (0, 0))
_deg_spec = pl.BlockSpec((NC, BLK, DEGW), lambda i: (0, i, 0))
_p_spec = pl.BlockSpec((NC, BLK, D), lambda i: (0, i, 0))
_b_spec = pl.BlockSpec((1, D), lambda i: (0, 0))

_g1_call = pl.pallas_call(
    _tc_g1_body,
    grid=(GRID,),
    in_specs=[_row_spec, _w_spec, _deg_spec],
    out_specs=_row_spec,
    out_shape=jax.ShapeDtypeStruct((N, D), jnp.float32),
)

_g2_call = pl.pallas_call(
    _tc_g2_body,
    grid=(GRID,),
    in_specs=[_p_spec, _row_spec, _deg_spec, _b_spec, _w_spec],
    out_specs=_row_spec,
    out_shape=jax.ShapeDtypeStruct((N, D), jnp.float32),
)

_out_call = pl.pallas_call(
    _tc_out_body,
    grid=(GRID,),
    in_specs=[_p_spec, _row_spec, _deg_spec, _b_spec],
    out_specs=_row_spec,
    out_shape=jax.ShapeDtypeStruct((N, D), jnp.float32),
)


def kernel(x, edge_index, W1, b1, W2, b2):
    packed = jnp.bitwise_or(
        edge_index[0], jnp.left_shift(edge_index[1], 16)
    ).reshape(NW, EPW)
    padv = jnp.full((NW, EPWPD - EPW), TRASH << 16, dtype=jnp.int32)
    pk4 = jnp.concatenate([packed, padv], axis=1).reshape(NW, NCHUNKD, 1, CHD)
    src4 = edge_index[0].reshape(NW * NCHUNK, 1, CH)
    dst4 = edge_index[1].reshape(NW * NCHUNK, 1, CH)
    b1r = b1.reshape(1, D)
    b2r = b2.reshape(1, D)

    deg16 = _sc_deg(pk4)
    g1 = _g1_call(x, W1, deg16)
    p1 = _sc_agg(g1, src4, dst4)
    g2 = _g2_call(p1, g1, deg16, b1r, W2)
    p2 = _sc_agg(g2, src4, dst4)
    out = _out_call(p2, g2, deg16, b2r)
    return out


# speedup vs baseline: 1.5471x; 1.0604x over previous
import functools

import jax
import jax.numpy as jnp
from jax import lax
from jax.experimental import pallas as pl
from jax.experimental.pallas import tpu as pltpu
from jax.experimental.pallas import tpu_sc as plsc

N = 10000
E = 320000
D = 128
NC = 2
NS = 16
NW = NC * NS
EPW = E // NW
CH = 125
NCHUNK = EPW // CH
CHD = 128
NCHUNKD = -(-EPW // CHD)
EPWPD = NCHUNKD * CHD
N2 = 10240
TRASH = N2 - 1
RPT = N2 // NS
ZR = 16
DEGW = 16
BLK = 400
GRID = N // BLK

_mesh = plsc.VectorSubcoreMesh(
    core_axis_name="c", subcore_axis_name="s", num_cores=NC, num_subcores=NS
)


def _unpack_chunk(pk_v, jc, st_s, st_d):
    def u(k, carry):
        v = pk_v[jc, 0, pl.ds(k * 16, 16)]
        if st_s is not None:
            st_s[pl.ds(k * 16, 16)] = jnp.bitwise_and(v, 0xFFFF)
        st_d[pl.ds(k * 16, 16)] = lax.shift_right_logical(v, 16)
        return carry

    lax.fori_loop(0, CHD // 16, u, 0)


@functools.partial(
    pl.kernel,
    out_type=jax.ShapeDtypeStruct((NC, N2, DEGW), jnp.float32),
    mesh=_mesh,
    scratch_types=[
        pltpu.VMEM((NCHUNKD, 1, CHD), jnp.int32),
        pltpu.VMEM((CHD,), jnp.int32),
        pltpu.VMEM((CHD,), jnp.int32),
        pltpu.VMEM((CHD, DEGW), jnp.float32),
        pltpu.VMEM((ZR, DEGW), jnp.float32),
        pltpu.VMEM_SHARED((N2, DEGW), jnp.float32),
        pltpu.SemaphoreType.DMA,
        pltpu.SemaphoreType.DMA,
    ],
)
def _sc_deg(pk_hbm, out_hbm, pk_v, std_a, std_b, ones_v, zer_v, acc, sem_a, sem_b):
    c = lax.axis_index("c")
    s = lax.axis_index("s")
    w = c * NS + s

    pltpu.async_copy(pk_hbm.at[w], pk_v, sem_a)

    def fill_ones(i, carry):
        ones_v[i, :] = jnp.ones((16,), jnp.float32)
        return carry

    lax.fori_loop(0, CHD, fill_ones, 0)

    def fill_zero(i, carry):
        zer_v[i, :] = jnp.zeros((16,), jnp.float32)
        return carry

    lax.fori_loop(0, ZR, fill_zero, 0)

    for k in range(RPT // ZR):
        pltpu.sync_copy(zer_v, acc.at[pl.ds(s * RPT + k * ZR, ZR)])

    pltpu.make_async_copy(pk_hbm.at[w], pk_v, sem_a).wait()
    plsc.subcore_barrier()

    _unpack_chunk(pk_v, 0, None, std_a)
    pltpu.async_copy(ones_v, acc.at[std_a], sem_a, add=True)
    _unpack_chunk(pk_v, 1, None, std_b)
    pltpu.async_copy(ones_v, acc.at[std_b], sem_b, add=True)

    def body(jj, carry):
        j = 2 * jj
        pltpu.make_async_copy(ones_v, acc.at[std_a], sem_a).wait()
        _unpack_chunk(pk_v, j + 2, None, std_a)
        pltpu.async_copy(ones_v, acc.at[std_a], sem_a, add=True)
        pltpu.make_async_copy(ones_v, acc.at[std_b], sem_b).wait()
        _unpack_chunk(pk_v, j + 3, None, std_b)
        pltpu.async_copy(ones_v, acc.at[std_b], sem_b, add=True)
        return carry

    lax.fori_loop(0, (NCHUNKD - 3) // 2, body, 0)
    pltpu.make_async_copy(ones_v, acc.at[std_a], sem_a).wait()
    _unpack_chunk(pk_v, NCHUNKD - 1, None, std_a)
    pltpu.async_copy(ones_v, acc.at[std_a], sem_a, add=True)
    pltpu.make_async_copy(ones_v, acc.at[std_a], sem_a).wait()
    pltpu.make_async_copy(ones_v, acc.at[std_b], sem_b).wait()
    plsc.subcore_barrier()
    pltpu.sync_copy(acc.at[pl.ds(s * RPT, RPT)], out_hbm.at[c, pl.ds(s * RPT, RPT)])


@functools.partial(
    pl.kernel,
    out_type=jax.ShapeDtypeStruct((NC, N2, D), jnp.float32),
    mesh=_mesh,
    scratch_types=[
        pltpu.VMEM((NCHUNK, 1, CH), jnp.int32),
        pltpu.VMEM((NCHUNK, 1, CH), jnp.int32),
        pltpu.VMEM((CH, D), jnp.float32),
        pltpu.VMEM((ZR, D), jnp.float32),
        pltpu.VMEM_SHARED((N2, D), jnp.float32),
        pltpu.SemaphoreType.DMA,
        pltpu.SemaphoreType.DMA,
    ],
)
def _sc_agg(g_hbm, src4_hbm, dst4_hbm, out_hbm, idxs_v, idxd_v,
            rows_v, zer_v, acc, sem_g, sem_i):
    c = lax.axis_index("c")
    s = lax.axis_index("s")
    w = c * NS + s

    pltpu.async_copy(src4_hbm.at[pl.ds(w * NCHUNK, NCHUNK)], idxs_v, sem_i)
    pltpu.async_copy(dst4_hbm.at[pl.ds(w * NCHUNK, NCHUNK)], idxd_v, sem_i)

    def fill_zero(i, carry):
        r = i // (D // 16)
        k = i - r * (D // 16)
        zer_v[r, pl.ds(k * 16, 16)] = jnp.zeros((16,), jnp.float32)
        return carry

    lax.fori_loop(0, ZR * (D // 16), fill_zero, 0)

    for k in range(RPT // ZR):
        pltpu.sync_copy(zer_v, acc.at[pl.ds(s * RPT + k * ZR, ZR)])

    pltpu.make_async_copy(src4_hbm.at[pl.ds(w * NCHUNK, NCHUNK)], idxs_v, sem_i).wait()
    pltpu.make_async_copy(dst4_hbm.at[pl.ds(w * NCHUNK, NCHUNK)], idxd_v, sem_i).wait()
    plsc.subcore_barrier()

    def body(j, carry):
        pltpu.async_copy(g_hbm.at[idxs_v.at[j, 0]], rows_v, sem_g).wait()
        pltpu.sync_copy(rows_v, acc.at[idxd_v.at[j, 0]], add=True)
        return carry

    lax.fori_loop(0, NCHUNK, body, 0)
    plsc.subcore_barrier()
    pltpu.sync_copy(acc.at[pl.ds(s * RPT, RPT)], out_hbm.at[c, pl.ds(s * RPT, RPT)])


def _dis(deg_ref):
    return lax.rsqrt(1.0 + deg_ref[0, :, 0:1] + deg_ref[1, :, 0:1])


def _tc_g1_body(x_ref, w_ref, deg_ref, o_ref):
    h = jnp.dot(x_ref[...], w_ref[...], preferred_element_type=jnp.float32)
    o_ref[...] = h * _dis(deg_ref)


def _tc_g2_body(p_ref, g_ref, deg_ref, b_ref, w_ref, o_ref):
    dis = _dis(deg_ref)
    ssum = p_ref[0] + p_ref[1] + g_ref[...]
    h = jnp.maximum(dis * ssum + b_ref[...], 0.0)
    o_ref[...] = jnp.dot(h, w_ref[...], preferred_element_type=jnp.float32) * dis


def _tc_out_body(p_ref, g_ref, deg_ref, b_ref, o_ref):
    dis = _dis(deg_ref)
    o_ref[...] = dis * (p_ref[0] + p_ref[1] + g_ref[...]) + b_ref[...]


_row_spec = pl.BlockSpec((BLK, D), lambda i: (i, 0))
_w_spec = pl.BlockSpec((D, D), lambda i: (0, 0))
_deg_spec = pl.BlockSpec((NC, BLK, DEGW), lambda i: (0, i, 0))
_p_spec = pl.BlockSpec((NC, BLK, D), lambda i: (0, i, 0))
_b_spec = pl.BlockSpec((1, D), lambda i: (0, 0))

_g1_call = pl.pallas_call(
    _tc_g1_body,
    grid=(GRID,),
    in_specs=[_row_spec, _w_spec, _deg_spec],
    out_specs=_row_spec,
    out_shape=jax.ShapeDtypeStruct((N, D), jnp.float32),
)

_g2_call = pl.pallas_call(
    _tc_g2_body,
    grid=(GRID,),
    in_specs=[_p_spec, _row_spec, _deg_spec, _b_spec, _w_spec],
    out_specs=_row_spec,
    out_shape=jax.ShapeDtypeStruct((N, D), jnp.float32),
)

_out_call = pl.pallas_call(
    _tc_out_body,
    grid=(GRID,),
    in_specs=[_p_spec, _row_spec, _deg_spec, _b_spec],
    out_specs=_row_spec,
    out_shape=jax.ShapeDtypeStruct((N, D), jnp.float32),
)


def kernel(x, edge_index, W1, b1, W2, b2):
    packed = jnp.bitwise_or(
        edge_index[0], jnp.left_shift(edge_index[1], 16)
    ).reshape(NW, EPW)
    padv = jnp.full((NW, EPWPD - EPW), TRASH << 16, dtype=jnp.int32)
    pk4 = jnp.concatenate([packed, padv], axis=1).reshape(NW, NCHUNKD, 1, CHD)
    src4 = edge_index[0].reshape(NW * NCHUNK, 1, CH)
    dst4 = edge_index[1].reshape(NW * NCHUNK, 1, CH)
    b1r = b1.reshape(1, D)
    b2r = b2.reshape(1, D)

    deg16 = _sc_deg(pk4)
    g1 = _g1_call(x, W1, deg16)
    p1 = _sc_agg(g1, src4, dst4)
    g2 = _g2_call(p1, g1, deg16, b1r, W2)
    p2 = _sc_agg(g2, src4, dst4)
    out = _out_call(p2, g2, deg16, b2r)
    return out


# speedup vs baseline: 1.6286x; 1.0527x over previous
import functools

import jax
import jax.numpy as jnp
from jax import lax
from jax.experimental import pallas as pl
from jax.experimental.pallas import tpu as pltpu
from jax.experimental.pallas import tpu_sc as plsc

N = 10000
E = 320000
D = 128
NC = 2
NS = 16
NW = NC * NS
EPW = E // NW
CH = 125
NCHUNK = EPW // CH
CHD = 128
NCHUNKD = -(-EPW // CHD)
EPWPD = NCHUNKD * CHD
N2 = 10240
TRASH = N2 - 1
RPT = N2 // NS
ZR = 16
DEGW = 16
BLK = 1000
GRID = N // BLK

_mesh = plsc.VectorSubcoreMesh(
    core_axis_name="c", subcore_axis_name="s", num_cores=NC, num_subcores=NS
)


def _unpack_chunk(pk_v, jc, st_s, st_d):
    def u(k, carry):
        v = pk_v[jc, 0, pl.ds(k * 16, 16)]
        if st_s is not None:
            st_s[pl.ds(k * 16, 16)] = jnp.bitwise_and(v, 0xFFFF)
        st_d[pl.ds(k * 16, 16)] = lax.shift_right_logical(v, 16)
        return carry

    lax.fori_loop(0, CHD // 16, u, 0)


@functools.partial(
    pl.kernel,
    out_type=jax.ShapeDtypeStruct((NC, N2, DEGW), jnp.float32),
    mesh=_mesh,
    scratch_types=[
        pltpu.VMEM((NCHUNKD, 1, CHD), jnp.int32),
        pltpu.VMEM((CHD,), jnp.int32),
        pltpu.VMEM((CHD,), jnp.int32),
        pltpu.VMEM((CHD, DEGW), jnp.float32),
        pltpu.VMEM((ZR, DEGW), jnp.float32),
        pltpu.VMEM_SHARED((N2, DEGW), jnp.float32),
        pltpu.SemaphoreType.DMA,
        pltpu.SemaphoreType.DMA,
    ],
)
def _sc_deg(pk_hbm, out_hbm, pk_v, std_a, std_b, ones_v, zer_v, acc, sem_a, sem_b):
    c = lax.axis_index("c")
    s = lax.axis_index("s")
    w = c * NS + s

    pltpu.async_copy(pk_hbm.at[w], pk_v, sem_a)

    def fill_ones(i, carry):
        ones_v[i, :] = jnp.ones((16,), jnp.float32)
        return carry

    lax.fori_loop(0, CHD, fill_ones, 0)

    def fill_zero(i, carry):
        zer_v[i, :] = jnp.zeros((16,), jnp.float32)
        return carry

    lax.fori_loop(0, ZR, fill_zero, 0)

    for k in range(RPT // ZR):
        pltpu.sync_copy(zer_v, acc.at[pl.ds(s * RPT + k * ZR, ZR)])

    pltpu.make_async_copy(pk_hbm.at[w], pk_v, sem_a).wait()
    plsc.subcore_barrier()

    _unpack_chunk(pk_v, 0, None, std_a)
    pltpu.async_copy(ones_v, acc.at[std_a], sem_a, add=True)
    _unpack_chunk(pk_v, 1, None, std_b)
    pltpu.async_copy(ones_v, acc.at[std_b], sem_b, add=True)

    def body(jj, carry):
        j = 2 * jj
        pltpu.make_async_copy(ones_v, acc.at[std_a], sem_a).wait()
        _unpack_chunk(pk_v, j + 2, None, std_a)
        pltpu.async_copy(ones_v, acc.at[std_a], sem_a, add=True)
        pltpu.make_async_copy(ones_v, acc.at[std_b], sem_b).wait()
        _unpack_chunk(pk_v, j + 3, None, std_b)
        pltpu.async_copy(ones_v, acc.at[std_b], sem_b, add=True)
        return carry

    lax.fori_loop(0, (NCHUNKD - 3) // 2, body, 0)
    pltpu.make_async_copy(ones_v, acc.at[std_a], sem_a).wait()
    _unpack_chunk(pk_v, NCHUNKD - 1, None, std_a)
    pltpu.async_copy(ones_v, acc.at[std_a], sem_a, add=True)
    pltpu.make_async_copy(ones_v, acc.at[std_a], sem_a).wait()
    pltpu.make_async_copy(ones_v, acc.at[std_b], sem_b).wait()
    plsc.subcore_barrier()
    pltpu.sync_copy(acc.at[pl.ds(s * RPT, RPT)], out_hbm.at[c, pl.ds(s * RPT, RPT)])


@functools.partial(
    pl.kernel,
    out_type=jax.ShapeDtypeStruct((NC, N2, D), jnp.float32),
    mesh=_mesh,
    scratch_types=[
        pltpu.VMEM((NCHUNK, 1, CH), jnp.int32),
        pltpu.VMEM((NCHUNK, 1, CH), jnp.int32),
        pltpu.VMEM((CH, D), jnp.float32),
        pltpu.VMEM((ZR, D), jnp.float32),
        pltpu.VMEM_SHARED((N2, D), jnp.float32),
        pltpu.SemaphoreType.DMA,
        pltpu.SemaphoreType.DMA,
    ],
)
def _sc_agg(g_hbm, src4_hbm, dst4_hbm, out_hbm, idxs_v, idxd_v,
            rows_v, zer_v, acc, sem_g, sem_i):
    c = lax.axis_index("c")
    s = lax.axis_index("s")
    w = c * NS + s

    pltpu.async_copy(src4_hbm.at[pl.ds(w * NCHUNK, NCHUNK)], idxs_v, sem_i)
    pltpu.async_copy(dst4_hbm.at[pl.ds(w * NCHUNK, NCHUNK)], idxd_v, sem_i)

    def fill_zero(i, carry):
        r = i // (D // 16)
        k = i - r * (D // 16)
        zer_v[r, pl.ds(k * 16, 16)] = jnp.zeros((16,), jnp.float32)
        return carry

    lax.fori_loop(0, ZR * (D // 16), fill_zero, 0)

    for k in range(RPT // ZR):
        pltpu.sync_copy(zer_v, acc.at[pl.ds(s * RPT + k * ZR, ZR)])

    pltpu.make_async_copy(src4_hbm.at[pl.ds(w * NCHUNK, NCHUNK)], idxs_v, sem_i).wait()
    pltpu.make_async_copy(dst4_hbm.at[pl.ds(w * NCHUNK, NCHUNK)], idxd_v, sem_i).wait()
    plsc.subcore_barrier()

    def body(j, carry):
        pltpu.async_copy(g_hbm.at[idxs_v.at[j, 0]], rows_v, sem_g).wait()
        pltpu.sync_copy(rows_v, acc.at[idxd_v.at[j, 0]], add=True)
        return carry

    lax.fori_loop(0, NCHUNK, body, 0)
    plsc.subcore_barrier()
    pltpu.sync_copy(acc.at[pl.ds(s * RPT, RPT)], out_hbm.at[c, pl.ds(s * RPT, RPT)])


def _dis(deg_ref):
    return lax.rsqrt(1.0 + deg_ref[0, :, 0:1] + deg_ref[1, :, 0:1])


def _tc_g1_body(x_ref, w_ref, deg_ref, o_ref):
    h = jnp.dot(x_ref[...], w_ref[...], preferred_element_type=jnp.float32)
    o_ref[...] = h * _dis(deg_ref)


def _tc_g2_body(p_ref, g_ref, deg_ref, b_ref, w_ref, o_ref):
    dis = _dis(deg_ref)
    ssum = p_ref[0] + p_ref[1] + g_ref[...]
    h = jnp.maximum(dis * ssum + b_ref[...], 0.0)
    o_ref[...] = jnp.dot(h, w_ref[...], preferred_element_type=jnp.float32) * dis


def _tc_out_body(p_ref, g_ref, deg_ref, b_ref, o_ref):
    dis = _dis(deg_ref)
    o_ref[...] = dis * (p_ref[0] + p_ref[1] + g_ref[...]) + b_ref[...]


_row_spec = pl.BlockSpec((BLK, D), lambda i: (i, 0))
_w_spec = pl.BlockSpec((D, D), lambda i: (0, 0))
_deg_spec = pl.BlockSpec((NC, BLK, DEGW), lambda i: (0, i, 0))
_p_spec = pl.BlockSpec((NC, BLK, D), lambda i: (0, i, 0))
_b_spec = pl.BlockSpec((1, D), lambda i: (0, 0))

_g1_call = pl.pallas_call(
    _tc_g1_body,
    grid=(GRID,),
    in_specs=[_row_spec, _w_spec, _deg_spec],
    out_specs=_row_spec,
    out_shape=jax.ShapeDtypeStruct((N, D), jnp.float32),
)

_g2_call = pl.pallas_call(
    _tc_g2_body,
    grid=(GRID,),
    in_specs=[_p_spec, _row_spec, _deg_spec, _b_spec, _w_spec],
    out_specs=_row_spec,
    out_shape=jax.ShapeDtypeStruct((N, D), jnp.float32),
)

_out_call = pl.pallas_call(
    _tc_out_body,
    grid=(GRID,),
    in_specs=[_p_spec, _row_spec, _deg_spec, _b_spec],
    out_specs=_row_spec,
    out_shape=jax.ShapeDtypeStruct((N, D), jnp.float32),
)


def kernel(x, edge_index, W1, b1, W2, b2):
    packed = jnp.bitwise_or(
        edge_index[0], jnp.left_shift(edge_index[1], 16)
    ).reshape(NW, EPW)
    padv = jnp.full((NW, EPWPD - EPW), TRASH << 16, dtype=jnp.int32)
    pk4 = jnp.concatenate([packed, padv], axis=1).reshape(NW, NCHUNKD, 1, CHD)
    src4 = edge_index[0].reshape(NW * NCHUNK, 1, CH)
    dst4 = edge_index[1].reshape(NW * NCHUNK, 1, CH)
    b1r = b1.reshape(1, D)
    b2r = b2.reshape(1, D)

    deg16 = _sc_deg(pk4)
    g1 = _g1_call(x, W1, deg16)
    p1 = _sc_agg(g1, src4, dst4)
    g2 = _g2_call(p1, g1, deg16, b1r, W2)
    p2 = _sc_agg(g2, src4, dst4)
    out = _out_call(p2, g2, deg16, b2r)
    return out


# speedup vs baseline: 1.6561x; 1.0169x over previous
import functools

import jax
import jax.numpy as jnp
from jax import lax
from jax.experimental import pallas as pl
from jax.experimental.pallas import tpu as pltpu
from jax.experimental.pallas import tpu_sc as plsc

N = 10000
E = 320000
D = 128
NC = 2
NS = 16
NW = NC * NS
EPW = E // NW
CH = 125
NCHUNK = EPW // CH
CHD = 128
NCHUNKD = -(-EPW // CHD)
EPWPD = NCHUNKD * CHD
N2 = 10240
TRASH = N2 - 1
RPT = N2 // NS
ZR = 16
DEGW = 16
BLK = 2000
GRID = N // BLK

_mesh = plsc.VectorSubcoreMesh(
    core_axis_name="c", subcore_axis_name="s", num_cores=NC, num_subcores=NS
)


def _unpack_chunk(pk_v, jc, st_s, st_d):
    def u(k, carry):
        v = pk_v[jc, 0, pl.ds(k * 16, 16)]
        if st_s is not None:
            st_s[pl.ds(k * 16, 16)] = jnp.bitwise_and(v, 0xFFFF)
        st_d[pl.ds(k * 16, 16)] = lax.shift_right_logical(v, 16)
        return carry

    lax.fori_loop(0, CHD // 16, u, 0)


@functools.partial(
    pl.kernel,
    out_type=jax.ShapeDtypeStruct((NC, N2, DEGW), jnp.float32),
    mesh=_mesh,
    scratch_types=[
        pltpu.VMEM((NCHUNKD, 1, CHD), jnp.int32),
        pltpu.VMEM((CHD,), jnp.int32),
        pltpu.VMEM((CHD,), jnp.int32),
        pltpu.VMEM((CHD, DEGW), jnp.float32),
        pltpu.VMEM((ZR, DEGW), jnp.float32),
        pltpu.VMEM_SHARED((N2, DEGW), jnp.float32),
        pltpu.SemaphoreType.DMA,
        pltpu.SemaphoreType.DMA,
    ],
)
def _sc_deg(pk_hbm, out_hbm, pk_v, std_a, std_b, ones_v, zer_v, acc, sem_a, sem_b):
    c = lax.axis_index("c")
    s = lax.axis_index("s")
    w = c * NS + s

    pltpu.async_copy(pk_hbm.at[w], pk_v, sem_a)

    def fill_ones(i, carry):
        ones_v[i, :] = jnp.ones((16,), jnp.float32)
        return carry

    lax.fori_loop(0, CHD, fill_ones, 0)

    def fill_zero(i, carry):
        zer_v[i, :] = jnp.zeros((16,), jnp.float32)
        return carry

    lax.fori_loop(0, ZR, fill_zero, 0)

    for k in range(RPT // ZR):
        pltpu.sync_copy(zer_v, acc.at[pl.ds(s * RPT + k * ZR, ZR)])

    pltpu.make_async_copy(pk_hbm.at[w], pk_v, sem_a).wait()
    plsc.subcore_barrier()

    _unpack_chunk(pk_v, 0, None, std_a)
    pltpu.async_copy(ones_v, acc.at[std_a], sem_a, add=True)
    _unpack_chunk(pk_v, 1, None, std_b)
    pltpu.async_copy(ones_v, acc.at[std_b], sem_b, add=True)

    def body(jj, carry):
        j = 2 * jj
        pltpu.make_async_copy(ones_v, acc.at[std_a], sem_a).wait()
        _unpack_chunk(pk_v, j + 2, None, std_a)
        pltpu.async_copy(ones_v, acc.at[std_a], sem_a, add=True)
        pltpu.make_async_copy(ones_v, acc.at[std_b], sem_b).wait()
        _unpack_chunk(pk_v, j + 3, None, std_b)
        pltpu.async_copy(ones_v, acc.at[std_b], sem_b, add=True)
        return carry

    lax.fori_loop(0, (NCHUNKD - 3) // 2, body, 0)
    pltpu.make_async_copy(ones_v, acc.at[std_a], sem_a).wait()
    _unpack_chunk(pk_v, NCHUNKD - 1, None, std_a)
    pltpu.async_copy(ones_v, acc.at[std_a], sem_a, add=True)
    pltpu.make_async_copy(ones_v, acc.at[std_a], sem_a).wait()
    pltpu.make_async_copy(ones_v, acc.at[std_b], sem_b).wait()
    plsc.subcore_barrier()
    pltpu.sync_copy(acc.at[pl.ds(s * RPT, RPT)], out_hbm.at[c, pl.ds(s * RPT, RPT)])


@functools.partial(
    pl.kernel,
    out_type=jax.ShapeDtypeStruct((NC, N2, D), jnp.float32),
    mesh=_mesh,
    scratch_types=[
        pltpu.VMEM((NCHUNK, 1, CH), jnp.int32),
        pltpu.VMEM((NCHUNK, 1, CH), jnp.int32),
        pltpu.VMEM((CH, D), jnp.float32),
        pltpu.VMEM((ZR, D), jnp.float32),
        pltpu.VMEM_SHARED((N2, D), jnp.float32),
        pltpu.SemaphoreType.DMA,
        pltpu.SemaphoreType.DMA,
    ],
)
def _sc_agg(g_hbm, src4_hbm, dst4_hbm, out_hbm, idxs_v, idxd_v,
            rows_v, zer_v, acc, sem_g, sem_i):
    c = lax.axis_index("c")
    s = lax.axis_index("s")
    w = c * NS + s

    pltpu.async_copy(src4_hbm.at[pl.ds(w * NCHUNK, NCHUNK)], idxs_v, sem_i)
    pltpu.async_copy(dst4_hbm.at[pl.ds(w * NCHUNK, NCHUNK)], idxd_v, sem_i)

    def fill_zero(i, carry):
        r = i // (D // 16)
        k = i - r * (D // 16)
        zer_v[r, pl.ds(k * 16, 16)] = jnp.zeros((16,), jnp.float32)
        return carry

    lax.fori_loop(0, ZR * (D // 16), fill_zero, 0)

    for k in range(RPT // ZR):
        pltpu.sync_copy(zer_v, acc.at[pl.ds(s * RPT + k * ZR, ZR)])

    pltpu.make_async_copy(src4_hbm.at[pl.ds(w * NCHUNK, NCHUNK)], idxs_v, sem_i).wait()
    pltpu.make_async_copy(dst4_hbm.at[pl.ds(w * NCHUNK, NCHUNK)], idxd_v, sem_i).wait()
    plsc.subcore_barrier()

    def body(j, carry):
        pltpu.async_copy(g_hbm.at[idxs_v.at[j, 0]], rows_v, sem_g).wait()
        pltpu.sync_copy(rows_v, acc.at[idxd_v.at[j, 0]], add=True)
        return carry

    lax.fori_loop(0, NCHUNK, body, 0)
    plsc.subcore_barrier()
    pltpu.sync_copy(acc.at[pl.ds(s * RPT, RPT)], out_hbm.at[c, pl.ds(s * RPT, RPT)])


def _dis(deg_ref):
    return lax.rsqrt(1.0 + deg_ref[0, :, 0:1] + deg_ref[1, :, 0:1])


def _tc_g1_body(x_ref, w_ref, deg_ref, o_ref):
    h = jnp.dot(x_ref[...], w_ref[...], preferred_element_type=jnp.float32)
    o_ref[...] = h * _dis(deg_ref)


def _tc_g2_body(p_ref, g_ref, deg_ref, b_ref, w_ref, o_ref):
    dis = _dis(deg_ref)
    ssum = p_ref[0] + p_ref[1] + g_ref[...]
    h = jnp.maximum(dis * ssum + b_ref[...], 0.0)
    o_ref[...] = jnp.dot(h, w_ref[...], preferred_element_type=jnp.float32) * dis


def _tc_out_body(p_ref, g_ref, deg_ref, b_ref, o_ref):
    dis = _dis(deg_ref)
    o_ref[...] = dis * (p_ref[0] + p_ref[1] + g_ref[...]) + b_ref[...]


_row_spec = pl.BlockSpec((BLK, D), lambda i: (i, 0))
_w_spec = pl.BlockSpec((D, D), lambda i: (0, 0))
_deg_spec = pl.BlockSpec((NC, BLK, DEGW), lambda i: (0, i, 0))
_p_spec = pl.BlockSpec((NC, BLK, D), lambda i: (0, i, 0))
_b_spec = pl.BlockSpec((1, D), lambda i: (0, 0))

_g1_call = pl.pallas_call(
    _tc_g1_body,
    grid=(GRID,),
    in_specs=[_row_spec, _w_spec, _deg_spec],
    out_specs=_row_spec,
    out_shape=jax.ShapeDtypeStruct((N, D), jnp.float32),
)

_g2_call = pl.pallas_call(
    _tc_g2_body,
    grid=(GRID,),
    in_specs=[_p_spec, _row_spec, _deg_spec, _b_spec, _w_spec],
    out_specs=_row_spec,
    out_shape=jax.ShapeDtypeStruct((N, D), jnp.float32),
)

_out_call = pl.pallas_call(
    _tc_out_body,
    grid=(GRID,),
    in_specs=[_p_spec, _row_spec, _deg_spec, _b_spec],
    out_specs=_row_spec,
    out_shape=jax.ShapeDtypeStruct((N, D), jnp.float32),
)


def kernel(x, edge_index, W1, b1, W2, b2):
    packed = jnp.bitwise_or(
        edge_index[0], jnp.left_shift(edge_index[1], 16)
    ).reshape(NW, EPW)
    padv = jnp.full((NW, EPWPD - EPW), TRASH << 16, dtype=jnp.int32)
    pk4 = jnp.concatenate([packed, padv], axis=1).reshape(NW, NCHUNKD, 1, CHD)
    src4 = edge_index[0].reshape(NW * NCHUNK, 1, CH)
    dst4 = edge_index[1].reshape(NW * NCHUNK, 1, CH)
    b1r = b1.reshape(1, D)
    b2r = b2.reshape(1, D)

    deg16 = _sc_deg(pk4)
    g1 = _g1_call(x, W1, deg16)
    p1 = _sc_agg(g1, src4, dst4)
    g2 = _g2_call(p1, g1, deg16, b1r, W2)
    p2 = _sc_agg(g2, src4, dst4)
    out = _out_call(p2, g2, deg16, b2r)
    return out
